# baseline (speedup 1.0000x reference)
import jax
import jax.numpy as jnp
from jax import lax
from jax.experimental import pallas as pl
from jax.experimental.pallas import tpu as pltpu
from jax.experimental.pallas import tpu_sc as plsc

NUM_TOKENS = 4096
SEQ = 200
DIM = 32
NUM_ROWS = 1000000
LANE = 16

NC = 2
NS = 16
NW = NC * NS

NB = 512
TGRID = (NUM_ROWS + NB - 1) // NB
FLAT_LINES = TGRID * 128
FLAT_ROWS = FLAT_LINES * 4


def _trelayout_body(tt_ref, out_ref):
    xb = tt_ref[...]
    eye = jnp.eye(DIM, dtype=jnp.float32)
    parts = []
    for k in range(NB // 128):
        parts.append(
            jax.lax.dot_general(
                xb[:, 128 * k:128 * (k + 1)], eye,
                (((0,), (0,)), ((), ())),
                precision=jax.lax.Precision.HIGHEST,
                preferred_element_type=jnp.float32,
            )
        )
    out_ref[...] = jnp.concatenate(parts, axis=1)


@jax.jit
def _trelayout(tt):
    return pl.pallas_call(
        _trelayout_body,
        out_shape=jax.ShapeDtypeStruct((FLAT_LINES, 128), jnp.float32),
        grid=(TGRID,),
        in_specs=[pl.BlockSpec((DIM, NB), lambda j: (0, j))],
        out_specs=pl.BlockSpec((128, 128), lambda j: (j, 0)),
    )(tt)


ROWS_PER_W = NUM_TOKENS // NW
NBUF = 8
NGROUP = ROWS_PER_W // NBUF


def _gather_body(idx_hbm, table_hbm, out_hbm, idx_all, idx2, rows, semg, sems):
    wid = lax.axis_index("s") * NC + lax.axis_index("c")
    r0 = wid * ROWS_PER_W

    pltpu.sync_copy(idx_hbm.at[pl.ds(r0, ROWS_PER_W)], idx_all)

    offs = tuple(range(0, SEQ - LANE + 1, LANE)) + (SEQ - LANE,)

    @plsc.parallel_loop(0, ROWS_PER_W, unroll=4)
    def _(i):
        for off in offs:
            r = idx_all[i, pl.ds(off, LANE)]
            rp = (
                (r & jnp.int32(-512))
                + ((r & jnp.int32(127)) << 2)
                + ((r >> 7) & jnp.int32(3))
            )
            idx2[i, pl.ds(off, LANE)] = rp

    def start_gather(i, b):
        pltpu.async_copy(table_hbm.at[idx2.at[i]], rows.at[b], semg.at[b])

    def wait_gather(b):
        pltpu.make_async_copy(
            table_hbm.at[pl.ds(0, SEQ)], rows.at[b], semg.at[b]
        ).wait()

    def start_store(i, b):
        pltpu.async_copy(rows.at[b], out_hbm.at[r0 + i], sems.at[b])

    def wait_store(b):
        pltpu.make_async_copy(rows.at[b], out_hbm.at[0], sems.at[b]).wait()

    for b in range(NBUF):
        start_gather(b, b)

    def turn(g, carry):
        i0 = g * NBUF
        for b in range(NBUF):
            wait_gather(b)
            start_store(i0 + b, b)
        for b in range(NBUF):
            wait_store(b)
            start_gather(i0 + NBUF + b, b)
        return carry

    lax.fori_loop(0, NGROUP - 1, turn, 0)

    i0 = (NGROUP - 1) * NBUF
    for b in range(NBUF):
        wait_gather(b)
        start_store(i0 + b, b)
    for b in range(NBUF):
        wait_store(b)


@jax.jit
def _embed(token_ids, table):
    mesh = plsc.VectorSubcoreMesh(core_axis_name="c", subcore_axis_name="s")
    return pl.kernel(
        _gather_body,
        out_type=jax.ShapeDtypeStruct((NUM_TOKENS, SEQ, DIM), jnp.float32),
        mesh=mesh,
        scratch_types=[
            pltpu.VMEM((ROWS_PER_W, SEQ), jnp.int32),
            pltpu.VMEM((ROWS_PER_W, SEQ), jnp.int32),
            pltpu.VMEM((NBUF, SEQ, DIM), jnp.float32),
            pltpu.SemaphoreType.DMA((NBUF,)),
            pltpu.SemaphoreType.DMA((NBUF,)),
        ],
        compiler_params=pltpu.CompilerParams(use_tc_tiling_on_sc=False),
    )(token_ids, table)


def kernel(token_ids, embeddings):
    flat = _trelayout(embeddings.T)
    table = flat.reshape(FLAT_ROWS, DIM)
    return _embed(jnp.asarray(token_ids, jnp.int32), table)


# speedup vs baseline: 1.7558x; 1.7558x over previous
import jax
import jax.numpy as jnp
from jax import lax
from jax.experimental import pallas as pl
from jax.experimental.pallas import tpu as pltpu
from jax.experimental.pallas import tpu_sc as plsc

NUM_TOKENS = 4096
SEQ = 200
DIM = 32
NUM_ROWS = 1000000
LANE = 16

NC = 2
NS = 16
NW = NC * NS

NB = 4096
TGRID = (NUM_ROWS + NB - 1) // NB
OUT_W = (NB // 128) * DIM
FLAT_LINES = TGRID * 128
FLAT_ROWS = FLAT_LINES * (OUT_W // DIM)


def _trelayout_body(tt_ref, out_ref):
    xb = tt_ref[...]
    eye = jnp.eye(DIM, dtype=jnp.float32)
    parts = []
    for k in range(NB // 128):
        parts.append(
            jax.lax.dot_general(
                xb[:, 128 * k:128 * (k + 1)], eye,
                (((0,), (0,)), ((), ())),
                precision=jax.lax.Precision.HIGHEST,
                preferred_element_type=jnp.float32,
            )
        )
    out_ref[...] = jnp.concatenate(parts, axis=1)


@jax.jit
def _trelayout(tt):
    return pl.pallas_call(
        _trelayout_body,
        out_shape=jax.ShapeDtypeStruct((FLAT_LINES, OUT_W), jnp.float32),
        grid=(TGRID,),
        in_specs=[pl.BlockSpec((DIM, NB), lambda j: (0, j))],
        out_specs=pl.BlockSpec((128, OUT_W), lambda j: (j, 0)),
    )(tt)


ROWS_PER_W = NUM_TOKENS // NW
NBUF = 8
NGROUP = ROWS_PER_W // NBUF


def _gather_body(idx_hbm, table_hbm, out_hbm, idx_all, idx2, rows, semg, sems):
    wid = lax.axis_index("s") * NC + lax.axis_index("c")
    r0 = wid * ROWS_PER_W

    pltpu.sync_copy(idx_hbm.at[pl.ds(r0, ROWS_PER_W)], idx_all)

    offs = tuple(range(0, SEQ - LANE + 1, LANE)) + (SEQ - LANE,)

    @plsc.parallel_loop(0, ROWS_PER_W, unroll=4)
    def _(i):
        for off in offs:
            r = idx_all[i, pl.ds(off, LANE)]
            rp = (
                (r & jnp.int32(-NB))
                + ((r & jnp.int32(127)) << 5)
                + ((r >> 7) & jnp.int32(NB // 128 - 1))
            )
            idx2[i, pl.ds(off, LANE)] = rp

    def start_gather(i, b):
        pltpu.async_copy(table_hbm.at[idx2.at[i]], rows.at[b], semg.at[b])

    def wait_gather(b):
        pltpu.make_async_copy(
            table_hbm.at[pl.ds(0, SEQ)], rows.at[b], semg.at[b]
        ).wait()

    def start_store(i, b):
        pltpu.async_copy(rows.at[b], out_hbm.at[r0 + i], sems.at[b])

    def wait_store(b):
        pltpu.make_async_copy(rows.at[b], out_hbm.at[0], sems.at[b]).wait()

    for b in range(NBUF):
        start_gather(b, b)

    def turn(g, carry):
        i0 = g * NBUF
        for b in range(NBUF):
            wait_gather(b)
            start_store(i0 + b, b)
        for b in range(NBUF):
            wait_store(b)
            start_gather(i0 + NBUF + b, b)
        return carry

    lax.fori_loop(0, NGROUP - 1, turn, 0)

    i0 = (NGROUP - 1) * NBUF
    for b in range(NBUF):
        wait_gather(b)
        start_store(i0 + b, b)
    for b in range(NBUF):
        wait_store(b)


@jax.jit
def _embed(token_ids, table):
    mesh = plsc.VectorSubcoreMesh(core_axis_name="c", subcore_axis_name="s")
    return pl.kernel(
        _gather_body,
        out_type=jax.ShapeDtypeStruct((NUM_TOKENS, SEQ, DIM), jnp.float32),
        mesh=mesh,
        scratch_types=[
            pltpu.VMEM((ROWS_PER_W, SEQ), jnp.int32),
            pltpu.VMEM((ROWS_PER_W, SEQ), jnp.int32),
            pltpu.VMEM((NBUF, SEQ, DIM), jnp.float32),
            pltpu.SemaphoreType.DMA((NBUF,)),
            pltpu.SemaphoreType.DMA((NBUF,)),
        ],
        compiler_params=pltpu.CompilerParams(use_tc_tiling_on_sc=False),
    )(token_ids, table)


def kernel(token_ids, embeddings):
    flat = _trelayout(embeddings.T)
    table = flat.reshape(FLAT_ROWS, DIM)
    return _embed(jnp.asarray(token_ids, jnp.int32), table)


# speedup vs baseline: 1.8464x; 1.0516x over previous
import jax
import jax.numpy as jnp
from jax import lax
from jax.experimental import pallas as pl
from jax.experimental.pallas import tpu as pltpu
from jax.experimental.pallas import tpu_sc as plsc

NUM_TOKENS = 4096
SEQ = 200
DIM = 32
NUM_ROWS = 1000000
LANE = 16

NC = 2
NS = 16
NW = NC * NS

NB = 16384
TGRID = (NUM_ROWS + NB - 1) // NB
OUT_W = (NB // 128) * DIM
FLAT_LINES = TGRID * 128
FLAT_ROWS = FLAT_LINES * (OUT_W // DIM)
KSHIFT = (NB // 128).bit_length() - 1


def _trelayout_body(tt_ref, out_ref):
    xb = tt_ref[...]
    eye = jnp.eye(DIM, dtype=jnp.float32)
    parts = []
    for k in range(NB // 128):
        parts.append(
            jax.lax.dot_general(
                xb[:, 128 * k:128 * (k + 1)], eye,
                (((0,), (0,)), ((), ())),
                precision=jax.lax.Precision.HIGHEST,
                preferred_element_type=jnp.float32,
            )
        )
    out_ref[...] = jnp.concatenate(parts, axis=1)


@jax.jit
def _trelayout(tt):
    return pl.pallas_call(
        _trelayout_body,
        out_shape=jax.ShapeDtypeStruct((FLAT_LINES, OUT_W), jnp.float32),
        grid=(TGRID,),
        in_specs=[pl.BlockSpec((DIM, NB), lambda j: (0, j))],
        out_specs=pl.BlockSpec((128, OUT_W), lambda j: (j, 0)),
    )(tt)


ROWS_PER_W = NUM_TOKENS // NW
NBUF = 8
NGROUP = ROWS_PER_W // NBUF


def _gather_body(idx_hbm, table_hbm, out_hbm, idx_all, idx2, rows, semg, sems):
    wid = lax.axis_index("s") * NC + lax.axis_index("c")
    r0 = wid * ROWS_PER_W

    pltpu.sync_copy(idx_hbm.at[pl.ds(r0, ROWS_PER_W)], idx_all)

    offs = tuple(range(0, SEQ - LANE + 1, LANE)) + (SEQ - LANE,)

    @plsc.parallel_loop(0, ROWS_PER_W, unroll=4)
    def _(i):
        for off in offs:
            r = idx_all[i, pl.ds(off, LANE)]
            rp = (
                (r & jnp.int32(-NB))
                + ((r & jnp.int32(127)) << KSHIFT)
                + ((r >> 7) & jnp.int32(NB // 128 - 1))
            )
            idx2[i, pl.ds(off, LANE)] = rp

    plsc.subcore_barrier()

    def start_gather(i, b):
        pltpu.async_copy(table_hbm.at[idx2.at[i]], rows.at[b], semg.at[b])

    def wait_gather(b):
        pltpu.make_async_copy(
            table_hbm.at[pl.ds(0, SEQ)], rows.at[b], semg.at[b]
        ).wait()

    def start_store(i, b):
        pltpu.async_copy(rows.at[b], out_hbm.at[r0 + i], sems.at[b])

    def wait_store(b):
        pltpu.make_async_copy(rows.at[b], out_hbm.at[0], sems.at[b]).wait()

    for b in range(NBUF):
        start_gather(b, b)

    def turn(g, carry):
        i0 = g * NBUF
        for b in range(NBUF):
            wait_gather(b)
            start_store(i0 + b, b)
        for b in range(NBUF):
            wait_store(b)
            start_gather(i0 + NBUF + b, b)
        return carry

    lax.fori_loop(0, NGROUP - 1, turn, 0)

    i0 = (NGROUP - 1) * NBUF
    for b in range(NBUF):
        wait_gather(b)
        start_store(i0 + b, b)
    for b in range(NBUF):
        wait_store(b)


@jax.jit
def _embed(token_ids, table):
    mesh = plsc.VectorSubcoreMesh(core_axis_name="c", subcore_axis_name="s")
    return pl.kernel(
        _gather_body,
        out_type=jax.ShapeDtypeStruct((NUM_TOKENS, SEQ, DIM), jnp.float32),
        mesh=mesh,
        scratch_types=[
            pltpu.VMEM((ROWS_PER_W, SEQ), jnp.int32),
            pltpu.VMEM((ROWS_PER_W, SEQ), jnp.int32),
            pltpu.VMEM((NBUF, SEQ, DIM), jnp.float32),
            pltpu.SemaphoreType.DMA((NBUF,)),
            pltpu.SemaphoreType.DMA((NBUF,)),
        ],
        compiler_params=pltpu.CompilerParams(use_tc_tiling_on_sc=False),
    )(token_ids, table)


def kernel(token_ids, embeddings):
    flat = _trelayout(embeddings.T)
    table = flat.reshape(FLAT_ROWS, DIM)
    return _embed(jnp.asarray(token_ids, jnp.int32), table)
